# baseline (speedup 1.0000x reference)
import functools

import jax
import jax.numpy as jnp
from jax import lax
from jax.experimental import pallas as pl
from jax.experimental.pallas import tpu as pltpu
from jax.experimental.pallas import tpu_sc as plsc

N_ROWS = 50000
D = 256
N_TOTAL = N_ROWS * D
NC = 2
NS = 16
NW = NC * NS
L = 16
VPR = D // L

R_SC = 7680
R_TC = N_ROWS - R_SC
ROWS_W = R_SC // NW
CH_ROWS = 120
NCH = ROWS_W // CH_ROWS
B_TC = 4232
N_STR = 5
G_TC = R_TC // B_TC // N_STR
SUBL = B_TC // 8


def _row_block(buf, r, vmax, vmin, vsum, vsq):
    for c in range(VPR):
        v = buf[r, pl.ds(c * L, L)]
        vmax = jnp.maximum(vmax, v)
        vmin = jnp.minimum(vmin, v)
        vsum = vsum + v
        vsq = vsq + v * v
    return vmax, vmin, vsum, vsq


def _sc_partials(x):
    mesh = plsc.VectorSubcoreMesh(core_axis_name="c", subcore_axis_name="s")

    @functools.partial(
        pl.kernel,
        mesh=mesh,
        out_type=jax.ShapeDtypeStruct((NW, 4 * L), jnp.float32),
        scratch_types=[
            pltpu.VMEM((CH_ROWS, D), jnp.float32),
            pltpu.VMEM((CH_ROWS, D), jnp.float32),
            pltpu.VMEM((4 * L,), jnp.float32),
            pltpu.SemaphoreType.DMA,
            pltpu.SemaphoreType.DMA,
        ],
    )
    def k(x_hbm, out_hbm, buf0, buf1, acc_v, sem0, sem1):
        wid = lax.axis_index("s") * NC + lax.axis_index("c")
        base = R_TC + wid * ROWS_W
        bufs = (buf0, buf1)
        sems = (sem0, sem1)

        copies = [None, None]
        copies[0] = pltpu.async_copy(
            x_hbm.at[pl.ds(base, CH_ROWS), :], buf0, sem0
        )

        vmax = jnp.full((L,), -jnp.inf, jnp.float32)
        vmin = jnp.full((L,), jnp.inf, jnp.float32)
        vsum = jnp.zeros((L,), jnp.float32)
        vsq = jnp.zeros((L,), jnp.float32)

        for g in range(NCH):
            if g + 1 < NCH:
                copies[(g + 1) % 2] = pltpu.async_copy(
                    x_hbm.at[pl.ds(base + (g + 1) * CH_ROWS, CH_ROWS), :],
                    bufs[(g + 1) % 2],
                    sems[(g + 1) % 2],
                )
            copies[g % 2].wait()
            buf = bufs[g % 2]

            def body(r, carry, buf=buf):
                return _row_block(buf, r, *carry)

            vmax, vmin, vsum, vsq = lax.fori_loop(
                0, CH_ROWS, body, (vmax, vmin, vsum, vsq)
            )

        acc_v[pl.ds(0, L)] = vmax
        acc_v[pl.ds(L, L)] = vmin
        acc_v[pl.ds(2 * L, L)] = vsum
        acc_v[pl.ds(3 * L, L)] = vsq
        pltpu.sync_copy(acc_v, out_hbm.at[wid])

    return k(x)


def _tc_partials_kernel(*refs):
    x_refs = refs[:N_STR]
    o_ref = refs[N_STR]
    mx_ref, mn_ref, sm_ref, sq_ref = refs[N_STR + 1:]
    i = pl.program_id(0)

    @pl.when(i == 0)
    def _():
        mx_ref[...] = jnp.full((8, D), -jnp.inf, jnp.float32)
        mn_ref[...] = jnp.full((8, D), jnp.inf, jnp.float32)
        sm_ref[...] = jnp.zeros((8, D), jnp.float32)
        sq_ref[...] = jnp.zeros((8, D), jnp.float32)

    for ref in x_refs:
        blk = ref[...].reshape(SUBL, 8, D)
        mx_ref[...] = jnp.maximum(mx_ref[...], jnp.max(blk, axis=0))
        mn_ref[...] = jnp.minimum(mn_ref[...], jnp.min(blk, axis=0))
        sm_ref[...] = sm_ref[...] + jnp.sum(blk, axis=0)
        sq_ref[...] = sq_ref[...] + jnp.sum(blk * blk, axis=0)

    @pl.when(i == G_TC - 1)
    def _():
        o_ref[0:8, :] = mx_ref[...]
        o_ref[8:16, :] = mn_ref[...]
        o_ref[16:24, :] = sm_ref[...]
        o_ref[24:32, :] = sq_ref[...]


def _make_spec(s):
    return pl.BlockSpec((B_TC, D), lambda i, s=s: (s * G_TC + i, 0))


def _tc_partials(x):
    return pl.pallas_call(
        _tc_partials_kernel,
        grid=(G_TC,),
        in_specs=[_make_spec(s) for s in range(N_STR)],
        out_specs=pl.BlockSpec((32, D), lambda i: (0, 0)),
        out_shape=jax.ShapeDtypeStruct((32, D), jnp.float32),
        scratch_shapes=[
            pltpu.VMEM((8, D), jnp.float32),
            pltpu.VMEM((8, D), jnp.float32),
            pltpu.VMEM((8, D), jnp.float32),
            pltpu.VMEM((8, D), jnp.float32),
        ],
    )(*([x] * N_STR))


def _finalize_kernel(sc_ref, tc_ref, wt_ref, b_ref, o_ref):
    p = sc_ref[...]
    t = tc_ref[...]
    gmax = jnp.maximum(jnp.max(p[:, 0:L]), jnp.max(t[0:8, :]))
    gmin = jnp.minimum(jnp.min(p[:, L:2 * L]), jnp.min(t[8:16, :]))
    gsum = jnp.sum(p[:, 2 * L:3 * L]) + jnp.sum(t[16:24, :])
    gsq = jnp.sum(p[:, 3 * L:4 * L]) + jnp.sum(t[24:32, :])
    n = jnp.float32(N_TOTAL)
    mean = gsum / n
    var = gsq / n - mean * mean
    std = jnp.sqrt(jnp.maximum(var, 0.0))
    wt = wt_ref[...]
    out = (
        gmax * wt[0:1, :]
        + gmin * wt[1:2, :]
        + mean * wt[2:3, :]
        + std * wt[3:4, :]
        + b_ref[...]
    )
    o_ref[...] = out


def kernel(x, W, b):
    sc_part = _sc_partials(x)
    tc_part = _tc_partials(x)
    out = pl.pallas_call(
        _finalize_kernel,
        out_shape=jax.ShapeDtypeStruct((1, D), jnp.float32),
    )(sc_part, tc_part, W.T, b.reshape(1, D))
    return out.reshape(D)


# speedup vs baseline: 1.0337x; 1.0337x over previous
import functools

import jax
import jax.numpy as jnp
from jax import lax
from jax.experimental import pallas as pl
from jax.experimental.pallas import tpu as pltpu
from jax.experimental.pallas import tpu_sc as plsc

N_ROWS = 50000
D = 256
N_TOTAL = N_ROWS * D
NC = 2
NS = 16
NW = NC * NS
L = 16
VPR = D // L

R_SC = 10240
R_TC = N_ROWS - R_SC
ROWS_W = R_SC // NW
CH_ROWS = 160
NCH = ROWS_W // CH_ROWS
B_TC = 3976
N_STR = 2
G_TC = R_TC // B_TC // N_STR
SUBL = B_TC // 8


def _row_block(buf, r, vmax, vmin, vsum, vsq):
    for c in range(VPR):
        v = buf[r, pl.ds(c * L, L)]
        vmax = jnp.maximum(vmax, v)
        vmin = jnp.minimum(vmin, v)
        vsum = vsum + v
        vsq = vsq + v * v
    return vmax, vmin, vsum, vsq


def _sc_partials(x):
    mesh = plsc.VectorSubcoreMesh(core_axis_name="c", subcore_axis_name="s")

    @functools.partial(
        pl.kernel,
        mesh=mesh,
        out_type=jax.ShapeDtypeStruct((NW, 4 * L), jnp.float32),
        scratch_types=[
            pltpu.VMEM((CH_ROWS, D), jnp.float32),
            pltpu.VMEM((CH_ROWS, D), jnp.float32),
            pltpu.VMEM((4 * L,), jnp.float32),
            pltpu.SemaphoreType.DMA,
            pltpu.SemaphoreType.DMA,
        ],
    )
    def k(x_hbm, out_hbm, buf0, buf1, acc_v, sem0, sem1):
        wid = lax.axis_index("s") * NC + lax.axis_index("c")
        base = R_TC + wid * ROWS_W
        bufs = (buf0, buf1)
        sems = (sem0, sem1)

        copies = [None, None]
        copies[0] = pltpu.async_copy(
            x_hbm.at[pl.ds(base, CH_ROWS), :], buf0, sem0
        )

        vmax = jnp.full((L,), -jnp.inf, jnp.float32)
        vmin = jnp.full((L,), jnp.inf, jnp.float32)
        vsum = jnp.zeros((L,), jnp.float32)
        vsq = jnp.zeros((L,), jnp.float32)

        for g in range(NCH):
            if g + 1 < NCH:
                copies[(g + 1) % 2] = pltpu.async_copy(
                    x_hbm.at[pl.ds(base + (g + 1) * CH_ROWS, CH_ROWS), :],
                    bufs[(g + 1) % 2],
                    sems[(g + 1) % 2],
                )
            copies[g % 2].wait()
            buf = bufs[g % 2]

            def body(r, carry, buf=buf):
                return _row_block(buf, r, *carry)

            vmax, vmin, vsum, vsq = lax.fori_loop(
                0, CH_ROWS, body, (vmax, vmin, vsum, vsq)
            )

        acc_v[pl.ds(0, L)] = vmax
        acc_v[pl.ds(L, L)] = vmin
        acc_v[pl.ds(2 * L, L)] = vsum
        acc_v[pl.ds(3 * L, L)] = vsq
        pltpu.sync_copy(acc_v, out_hbm.at[wid])

    return k(x)


def _tc_partials_kernel(*refs):
    x_refs = refs[:N_STR]
    o_ref = refs[N_STR]
    mx_ref, mn_ref, sm_ref, sq_ref = refs[N_STR + 1:]
    i = pl.program_id(0)

    @pl.when(i == 0)
    def _():
        mx_ref[...] = jnp.full((8, D), -jnp.inf, jnp.float32)
        mn_ref[...] = jnp.full((8, D), jnp.inf, jnp.float32)
        sm_ref[...] = jnp.zeros((8, D), jnp.float32)
        sq_ref[...] = jnp.zeros((8, D), jnp.float32)

    for ref in x_refs:
        blk = ref[...].reshape(SUBL, 8, D)
        mx_ref[...] = jnp.maximum(mx_ref[...], jnp.max(blk, axis=0))
        mn_ref[...] = jnp.minimum(mn_ref[...], jnp.min(blk, axis=0))
        sm_ref[...] = sm_ref[...] + jnp.sum(blk, axis=0)
        sq_ref[...] = sq_ref[...] + jnp.sum(blk * blk, axis=0)

    @pl.when(i == G_TC - 1)
    def _():
        o_ref[0:8, :] = mx_ref[...]
        o_ref[8:16, :] = mn_ref[...]
        o_ref[16:24, :] = sm_ref[...]
        o_ref[24:32, :] = sq_ref[...]


def _make_spec(s):
    return pl.BlockSpec((B_TC, D), lambda i, s=s: (s * G_TC + i, 0))


def _tc_partials(x):
    return pl.pallas_call(
        _tc_partials_kernel,
        grid=(G_TC,),
        in_specs=[_make_spec(s) for s in range(N_STR)],
        out_specs=pl.BlockSpec((32, D), lambda i: (0, 0)),
        out_shape=jax.ShapeDtypeStruct((32, D), jnp.float32),
        scratch_shapes=[
            pltpu.VMEM((8, D), jnp.float32),
            pltpu.VMEM((8, D), jnp.float32),
            pltpu.VMEM((8, D), jnp.float32),
            pltpu.VMEM((8, D), jnp.float32),
        ],
    )(*([x] * N_STR))


def _finalize_kernel(sc_ref, tc_ref, wt_ref, b_ref, o_ref):
    p = sc_ref[...]
    t = tc_ref[...]
    gmax = jnp.maximum(jnp.max(p[:, 0:L]), jnp.max(t[0:8, :]))
    gmin = jnp.minimum(jnp.min(p[:, L:2 * L]), jnp.min(t[8:16, :]))
    gsum = jnp.sum(p[:, 2 * L:3 * L]) + jnp.sum(t[16:24, :])
    gsq = jnp.sum(p[:, 3 * L:4 * L]) + jnp.sum(t[24:32, :])
    n = jnp.float32(N_TOTAL)
    mean = gsum / n
    var = gsq / n - mean * mean
    std = jnp.sqrt(jnp.maximum(var, 0.0))
    wt = wt_ref[...]
    out = (
        gmax * wt[0:1, :]
        + gmin * wt[1:2, :]
        + mean * wt[2:3, :]
        + std * wt[3:4, :]
        + b_ref[...]
    )
    o_ref[...] = out


def kernel(x, W, b):
    sc_part = _sc_partials(x)
    tc_part = _tc_partials(x)
    out = pl.pallas_call(
        _finalize_kernel,
        out_shape=jax.ShapeDtypeStruct((1, D), jnp.float32),
    )(sc_part, tc_part, W.T, b.reshape(1, D))
    return out.reshape(D)
